# baseline (speedup 1.0000x reference)
import functools

import jax
import jax.numpy as jnp
from jax import lax
from jax.experimental import pallas as pl
from jax.experimental.pallas import tpu as pltpu
from jax.experimental.pallas import tpu_sc as plsc

NU = 52643
NI = 91599
D = 32
B = 16384
NC = 2
NS = 16
NW = NC * NS
BPW = B // NW
CH = 128
NCH = BPW // CH
L = 16
NBUF = 4

NBU_FULL = NU // 128
WU_TAIL = NU - NBU_FULL * 128
NBU = NBU_FULL + 1
NBI_FULL = NI // 128
WI_TAIL = NI - NBI_FULL * 128
NBI = NBI_FULL + 1
ROW_I0 = NBU * 32
M = (NBU + NBI) * 32


def _transpose_blocks(pairs, width, lane):
    lcols, srows, sqbs = [], [], []
    for bc in range(0, 128, L):
        c = bc + lane
        lcols.append(c if width is None
                     else jnp.minimum(c, width - 1))
        srows.append(lax.shift_right_logical(c, 2))
        sqbs.append(lax.shift_left(jnp.bitwise_and(c, 3), 5))
    pending = []
    for bd in (0, 16):
        for r in range(L):
            lrow = bd + jnp.bitwise_and(lane + r, L - 1)
            cur = []
            for tv, ov in pairs:
                for t in range(8):
                    cur.append((ov, srows[t], sqbs[t] + lrow,
                                plsc.load_gather(tv, [lrow, lcols[t]])))
            for ov, srow, scol, v in pending:
                plsc.store_scatter(ov, [srow, scol], v)
            pending = cur
    for ov, srow, scol, v in pending:
        plsc.store_scatter(ov, [srow, scol], v)


def _detr_body(ut_hbm, it_hbm, lane_hbm, scr_hbm, tvs, ovs, tvu, tvi,
               lane_v, sis, sos):
    wid = lax.axis_index("s") * NC + lax.axis_index("c")
    pltpu.sync_copy(lane_hbm, lane_v)
    lane = lane_v[pl.ds(0, L)]

    def run_blocks(t_hbm, n_full, row_base):
        n_grp = (((n_full + NW - 1) // NW) + NBUF - 1) // NBUF

        def bidx(i):
            return jnp.minimum(wid + i * NW, n_full - 1)

        def in_dma(i, t):
            pltpu.async_copy(
                t_hbm.at[:, pl.ds(bidx(i) * 128, 128)], tvs[t], sis[t])

        def wait_in(t):
            pltpu.make_async_copy(
                t_hbm.at[:, pl.ds(0, 128)], tvs[t], sis[t]).wait()

        def out_dma(i, t):
            pltpu.async_copy(
                ovs[t],
                scr_hbm.at[pl.ds(row_base + bidx(i) * 32, 32), :],
                sos[t])

        def wait_out(t):
            pltpu.make_async_copy(
                ovs[t], scr_hbm.at[pl.ds(0, 32), :], sos[t]).wait()

        for t in range(NBUF):
            in_dma(t, t)

        def grp(k, _):
            for p in range(NBUF // 2):
                ts = (2 * p, 2 * p + 1)
                for t in ts:
                    wait_in(t)

                @pl.when(k > 0)
                def _():
                    for t in ts:
                        wait_out(t)

                _transpose_blocks(
                    [(tvs[t], ovs[t]) for t in ts], None, lane)
                for t in ts:
                    out_dma(k * NBUF + t, t)

                    @pl.when(k < n_grp - 1)
                    def _():
                        in_dma(k * NBUF + t + NBUF, t)
            return 0

        lax.fori_loop(0, n_grp, grp, 0)
        for t in range(NBUF):
            wait_out(t)

    run_blocks(ut_hbm, NBU_FULL, 0)
    run_blocks(it_hbm, NBI_FULL, ROW_I0)

    @pl.when(wid == 0)
    def _():
        pltpu.sync_copy(ut_hbm.at[:, pl.ds(NBU_FULL * 128, WU_TAIL)], tvu)
        _transpose_blocks([(tvu, ovs[0])], WU_TAIL, lane)
        pltpu.sync_copy(ovs[0], scr_hbm.at[pl.ds(NBU_FULL * 32, 32), :])

    @pl.when(wid == 1)
    def _():
        pltpu.sync_copy(it_hbm.at[:, pl.ds(NBI_FULL * 128, WI_TAIL)], tvi)
        _transpose_blocks([(tvi, ovs[1])], WI_TAIL, lane)
        pltpu.sync_copy(
            ovs[1], scr_hbm.at[pl.ds(ROW_I0 + NBI_FULL * 32, 32), :])


def _gdot_body(scr_hbm, user_hbm, item_i_hbm, item_j_hbm, lane_hbm,
               out_i_hbm, out_j_hbm,
               uidx_v, iidx_v, jidx_v, urow_v, irow_v, jrow_v,
               bufs, oi_v, oj_v, lane_v, sgs):
    wid = lax.axis_index("s") * NC + lax.axis_index("c")
    base = wid * BPW
    pltpu.sync_copy(lane_hbm, lane_v)
    lane = lane_v[pl.ds(0, L)]

    pltpu.sync_copy(user_hbm.at[pl.ds(base, BPW)], uidx_v)
    pltpu.sync_copy(item_i_hbm.at[pl.ds(base, BPW)], iidx_v)
    pltpu.sync_copy(item_j_hbm.at[pl.ds(base, BPW)], jidx_v)

    def rowidx(t, _):
        urow_v[pl.ds(t * L, L)] = lax.shift_right_logical(
            uidx_v[pl.ds(t * L, L)], 2)
        irow_v[pl.ds(t * L, L)] = lax.shift_right_logical(
            iidx_v[pl.ds(t * L, L)], 2) + ROW_I0
        jrow_v[pl.ds(t * L, L)] = lax.shift_right_logical(
            jidx_v[pl.ds(t * L, L)], 2) + ROW_I0
        return 0

    lax.fori_loop(0, BPW // L, rowidx, 0)

    def fire(c, par):
        u_v, vi_v, vj_v = bufs[par]
        pltpu.async_copy(
            scr_hbm.at[urow_v.at[pl.ds(c * CH, CH)]], u_v, sgs[par])
        pltpu.async_copy(
            scr_hbm.at[irow_v.at[pl.ds(c * CH, CH)]], vi_v, sgs[par])
        pltpu.async_copy(
            scr_hbm.at[jrow_v.at[pl.ds(c * CH, CH)]], vj_v, sgs[par])

    def drain(par):
        u_v, vi_v, vj_v = bufs[par]
        for v in (u_v, vi_v, vj_v):
            pltpu.make_async_copy(
                scr_hbm.at[urow_v.at[pl.ds(0, CH)]], v, sgs[par]).wait()

    def compute(c, par):
        u_v, vi_v, vj_v = bufs[par]

        def group(g, _):
            off = c * CH + g * L
            rows = g * L + lane
            qu = lax.shift_left(
                jnp.bitwise_and(uidx_v[pl.ds(off, L)], 3), 5)
            qi = lax.shift_left(
                jnp.bitwise_and(iidx_v[pl.ds(off, L)], 3), 5)
            qj = lax.shift_left(
                jnp.bitwise_and(jidx_v[pl.ds(off, L)], 3), 5)
            acc_i = [jnp.zeros((L,), jnp.float32) for _ in range(2)]
            acc_j = [jnp.zeros((L,), jnp.float32) for _ in range(2)]
            pending = []
            for d0 in range(0, D, 4):
                loads = []
                for d in range(d0, d0 + 4):
                    dd = jnp.bitwise_and(lane + d, D - 1)
                    uc = plsc.load_gather(u_v, [rows, qu + dd])
                    ic = plsc.load_gather(vi_v, [rows, qi + dd])
                    jc = plsc.load_gather(vj_v, [rows, qj + dd])
                    loads.append((uc, ic, jc))
                for k, (uc, ic, jc) in enumerate(pending):
                    acc_i[k % 2] = acc_i[k % 2] + uc * ic
                    acc_j[k % 2] = acc_j[k % 2] + uc * jc
                pending = loads
            for k, (uc, ic, jc) in enumerate(pending):
                acc_i[k % 2] = acc_i[k % 2] + uc * ic
                acc_j[k % 2] = acc_j[k % 2] + uc * jc
            oi_v[pl.ds(off, L)] = acc_i[0] + acc_i[1]
            oj_v[pl.ds(off, L)] = acc_j[0] + acc_j[1]
            return 0

        lax.fori_loop(0, CH // L, group, 0)

    fire(0, 0)

    def pair(p, _):
        for par in range(2):
            c = p * 2 + par

            @pl.when(c + 1 < NCH)
            def _():
                fire(c + 1, 1 - par)
            drain(par)
            compute(c, par)
        return 0

    lax.fori_loop(0, NCH // 2, pair, 0)

    pltpu.sync_copy(oi_v, out_i_hbm.at[pl.ds(base, BPW)])
    pltpu.sync_copy(oj_v, out_j_hbm.at[pl.ds(base, BPW)])


@jax.jit
def _bpr(user, item_i, item_j, embed_user_weight, embed_item_weight):
    mesh = plsc.VectorSubcoreMesh(core_axis_name="c", subcore_axis_name="s")
    params = pltpu.CompilerParams(needs_layout_passes=False)

    def detr_body(ut, it, lane_hbm, scr, *scratch):
        tvs = scratch[0:NBUF]
        ovs = scratch[NBUF:2 * NBUF]
        tvu, tvi = scratch[2 * NBUF], scratch[2 * NBUF + 1]
        lane_v = scratch[2 * NBUF + 2]
        sis = scratch[2 * NBUF + 3:2 * NBUF + 3 + NBUF]
        sos = scratch[2 * NBUF + 3 + NBUF:2 * NBUF + 3 + 2 * NBUF]
        _detr_body(ut, it, lane_hbm, scr, tvs, ovs, tvu, tvi, lane_v, sis, sos)

    detr = pl.kernel(
        detr_body,
        out_type=jax.ShapeDtypeStruct((M, 128), jnp.float32),
        mesh=mesh,
        compiler_params=params,
        scratch_types=(
            [pltpu.VMEM((D, 128), jnp.float32) for _ in range(NBUF)]
            + [pltpu.VMEM((32, 128), jnp.float32) for _ in range(NBUF)]
            + [pltpu.VMEM((D, WU_TAIL), jnp.float32),
               pltpu.VMEM((D, WI_TAIL), jnp.float32),
               pltpu.VMEM((L,), jnp.int32)]
            + [pltpu.SemaphoreType.DMA for _ in range(2 * NBUF)]
        ),
    )

    def gdot_body(scr, u, ii, ij, lane_hbm, oi, oj, *scratch):
        uidx_v, iidx_v, jidx_v, urow_v, irow_v, jrow_v = scratch[0:6]
        bufs = (scratch[6:9], scratch[9:12])
        oi_v, oj_v = scratch[12], scratch[13]
        lane_v = scratch[14]
        sgs = scratch[15:17]
        _gdot_body(scr, u, ii, ij, lane_hbm, oi, oj,
                   uidx_v, iidx_v, jidx_v, urow_v, irow_v, jrow_v,
                   bufs, oi_v, oj_v, lane_v, sgs)

    gdot = pl.kernel(
        gdot_body,
        out_type=(jax.ShapeDtypeStruct((B,), jnp.float32),
                  jax.ShapeDtypeStruct((B,), jnp.float32)),
        mesh=mesh,
        compiler_params=params,
        scratch_types=(
            [pltpu.VMEM((BPW,), jnp.int32) for _ in range(6)]
            + [pltpu.VMEM((CH, 128), jnp.float32) for _ in range(6)]
            + [pltpu.VMEM((BPW,), jnp.float32) for _ in range(2)]
            + [pltpu.VMEM((L,), jnp.int32)]
            + [pltpu.SemaphoreType.DMA for _ in range(2)]
        ),
    )
    lane_arr = jnp.arange(L, dtype=jnp.int32)
    scratch = detr(embed_user_weight.T, embed_item_weight.T, lane_arr)
    return gdot(scratch, user, item_i, item_j, lane_arr)


def kernel(user, item_i, item_j, embed_user_weight, embed_item_weight):
    return _bpr(user.astype(jnp.int32), item_i.astype(jnp.int32),
                item_j.astype(jnp.int32), embed_user_weight, embed_item_weight)


# speedup vs baseline: 1.2493x; 1.2493x over previous
import functools

import jax
import jax.numpy as jnp
from jax import lax
from jax.experimental import pallas as pl
from jax.experimental.pallas import tpu as pltpu
from jax.experimental.pallas import tpu_sc as plsc

NU = 52643
NI = 91599
D = 32
B = 16384
NC = 2
NS = 16
NW = NC * NS
BPW = B // NW
CH = 128
NCH = BPW // CH
L = 16
NBUF = 2

NBU_FULL = NU // 128
WU_TAIL = NU - NBU_FULL * 128
NBU = NBU_FULL + 1
NBI_FULL = NI // 128
WI_TAIL = NI - NBI_FULL * 128
NBI = NBI_FULL + 1
ROW_I0 = NBU * 32
M = (NBU + NBI) * 32


def _transpose_blocks(pairs, width, lane):
    lcols, srows, sqbs = [], [], []
    for bc in range(0, 128, L):
        c = bc + lane
        lcols.append(c if width is None
                     else jnp.minimum(c, width - 1))
        srows.append(lax.shift_right_logical(c, 2))
        sqbs.append(lax.shift_left(jnp.bitwise_and(c, 3), 5))
    entries = [(tv, ov,
                [lc + coff for lc in lcols],
                [sr + roff for sr in srows])
               for tv, ov, coff, roff in pairs]

    def wave(rr):
        lrow = jnp.bitwise_and(rr, L) + jnp.bitwise_and(lane + rr, L - 1)
        scols = [sqbs[t] + lrow for t in range(8)]
        out = []
        for tv, ov, lcs, srs in entries:
            for t in range(8):
                out.append((ov, srs[t], scols[t],
                            plsc.load_gather(tv, [lrow, lcs[t]])))
        return out

    def wbody(w, _):
        cur = wave(w * 2) + wave(w * 2 + 1)
        for ov, srow, scol, v in cur:
            plsc.store_scatter(ov, [srow, scol], v)
        return 0

    lax.fori_loop(0, L, wbody, 0)


def _detr_body(ut_hbm, it_hbm, lane_hbm, scr_hbm, tvs, ovs, tvu, tvi,
               lane_v, sis, sos):
    wid = lax.axis_index("s") * NC + lax.axis_index("c")
    pltpu.sync_copy(lane_hbm, lane_v)
    lane = lane_v[pl.ds(0, L)]

    def run_blocks(t_hbm, n_full, row_base):
        n_per = (n_full + NW - 1) // NW
        n_pairs = (n_per + 1) // 2
        n_body = (n_pairs + 1) // 2
        n_slots = 2 * n_body

        def bidx(p):
            return jnp.minimum(wid * n_per + 2 * p, n_full - 2)

        def in_dma(p, t):
            pltpu.async_copy(
                t_hbm.at[:, pl.ds(bidx(p) * 128, 256)], tvs[t], sis[t])

        def wait_in(t):
            pltpu.make_async_copy(
                t_hbm.at[:, pl.ds(0, 256)], tvs[t], sis[t]).wait()

        def out_dma(p, t):
            pltpu.async_copy(
                ovs[t],
                scr_hbm.at[pl.ds(row_base + bidx(p) * 32, 64), :],
                sos[t])

        def wait_out(t):
            pltpu.make_async_copy(
                ovs[t], scr_hbm.at[pl.ds(0, 64), :], sos[t]).wait()

        in_dma(0, 0)
        in_dma(1, 1)

        def grp(k, _):
            for par in range(2):
                p = 2 * k + par
                wait_in(par)

                @pl.when(k > 0)
                def _():
                    wait_out(par)

                _transpose_blocks(
                    [(tvs[par], ovs[par], 0, 0),
                     (tvs[par], ovs[par], 128, 32)], None, lane)
                out_dma(p, par)

                @pl.when(p + 2 < n_slots)
                def _():
                    in_dma(p + 2, par)
            return 0

        lax.fori_loop(0, n_body, grp, 0)
        for t in range(NBUF):
            wait_out(t)

    run_blocks(ut_hbm, NBU_FULL, 0)
    run_blocks(it_hbm, NBI_FULL, ROW_I0)

    @pl.when(wid == 0)
    def _():
        pltpu.sync_copy(ut_hbm.at[:, pl.ds(NBU_FULL * 128, WU_TAIL)], tvu)
        _transpose_blocks([(tvu, ovs[0], 0, 0)], WU_TAIL, lane)
        pltpu.sync_copy(ovs[0].at[pl.ds(0, 32), :],
                        scr_hbm.at[pl.ds(NBU_FULL * 32, 32), :])

    @pl.when(wid == 1)
    def _():
        pltpu.sync_copy(it_hbm.at[:, pl.ds(NBI_FULL * 128, WI_TAIL)], tvi)
        _transpose_blocks([(tvi, ovs[1], 0, 0)], WI_TAIL, lane)
        pltpu.sync_copy(
            ovs[1].at[pl.ds(0, 32), :],
            scr_hbm.at[pl.ds(ROW_I0 + NBI_FULL * 32, 32), :])


def _gdot_body(scr_hbm, user_hbm, item_i_hbm, item_j_hbm, lane_hbm,
               out_i_hbm, out_j_hbm,
               uidx_v, iidx_v, jidx_v, urow_v, irow_v, jrow_v,
               bufs, oi_v, oj_v, lane_v, sgs):
    wid = lax.axis_index("s") * NC + lax.axis_index("c")
    base = wid * BPW
    pltpu.sync_copy(lane_hbm, lane_v)
    lane = lane_v[pl.ds(0, L)]

    pltpu.sync_copy(user_hbm.at[pl.ds(base, BPW)], uidx_v)
    pltpu.sync_copy(item_i_hbm.at[pl.ds(base, BPW)], iidx_v)
    pltpu.sync_copy(item_j_hbm.at[pl.ds(base, BPW)], jidx_v)

    def rowidx(t, _):
        urow_v[pl.ds(t * L, L)] = lax.shift_right_logical(
            uidx_v[pl.ds(t * L, L)], 2)
        irow_v[pl.ds(t * L, L)] = lax.shift_right_logical(
            iidx_v[pl.ds(t * L, L)], 2) + ROW_I0
        jrow_v[pl.ds(t * L, L)] = lax.shift_right_logical(
            jidx_v[pl.ds(t * L, L)], 2) + ROW_I0
        return 0

    lax.fori_loop(0, BPW // L, rowidx, 0)

    def fire(c, par):
        u_v, vi_v, vj_v = bufs[par]
        pltpu.async_copy(
            scr_hbm.at[urow_v.at[pl.ds(c * CH, CH)]], u_v, sgs[par])
        pltpu.async_copy(
            scr_hbm.at[irow_v.at[pl.ds(c * CH, CH)]], vi_v, sgs[par])
        pltpu.async_copy(
            scr_hbm.at[jrow_v.at[pl.ds(c * CH, CH)]], vj_v, sgs[par])

    def drain(par):
        u_v, vi_v, vj_v = bufs[par]
        for v in (u_v, vi_v, vj_v):
            pltpu.make_async_copy(
                scr_hbm.at[urow_v.at[pl.ds(0, CH)]], v, sgs[par]).wait()

    def compute(c, par):
        u_v, vi_v, vj_v = bufs[par]

        def group(g, _):
            off = c * CH + g * L
            rows = g * L + lane
            qu = lax.shift_left(
                jnp.bitwise_and(uidx_v[pl.ds(off, L)], 3), 5)
            qi = lax.shift_left(
                jnp.bitwise_and(iidx_v[pl.ds(off, L)], 3), 5)
            qj = lax.shift_left(
                jnp.bitwise_and(jidx_v[pl.ds(off, L)], 3), 5)
            acc_i = [jnp.zeros((L,), jnp.float32) for _ in range(2)]
            acc_j = [jnp.zeros((L,), jnp.float32) for _ in range(2)]
            pending = []
            for d0 in range(0, D, 4):
                loads = []
                for d in range(d0, d0 + 4):
                    dd = jnp.bitwise_and(lane + d, D - 1)
                    uc = plsc.load_gather(u_v, [rows, qu + dd])
                    ic = plsc.load_gather(vi_v, [rows, qi + dd])
                    jc = plsc.load_gather(vj_v, [rows, qj + dd])
                    loads.append((uc, ic, jc))
                for k, (uc, ic, jc) in enumerate(pending):
                    acc_i[k % 2] = acc_i[k % 2] + uc * ic
                    acc_j[k % 2] = acc_j[k % 2] + uc * jc
                pending = loads
            for k, (uc, ic, jc) in enumerate(pending):
                acc_i[k % 2] = acc_i[k % 2] + uc * ic
                acc_j[k % 2] = acc_j[k % 2] + uc * jc
            oi_v[pl.ds(off, L)] = acc_i[0] + acc_i[1]
            oj_v[pl.ds(off, L)] = acc_j[0] + acc_j[1]
            return 0

        lax.fori_loop(0, CH // L, group, 0)

    fire(0, 0)

    def pair(p, _):
        for par in range(2):
            c = p * 2 + par

            @pl.when(c + 1 < NCH)
            def _():
                fire(c + 1, 1 - par)
            drain(par)
            compute(c, par)
        return 0

    lax.fori_loop(0, NCH // 2, pair, 0)

    pltpu.sync_copy(oi_v, out_i_hbm.at[pl.ds(base, BPW)])
    pltpu.sync_copy(oj_v, out_j_hbm.at[pl.ds(base, BPW)])


@jax.jit
def _bpr(user, item_i, item_j, embed_user_weight, embed_item_weight):
    mesh = plsc.VectorSubcoreMesh(core_axis_name="c", subcore_axis_name="s")
    params = pltpu.CompilerParams(needs_layout_passes=False)

    def detr_body(ut, it, lane_hbm, scr, *scratch):
        tvs = scratch[0:NBUF]
        ovs = scratch[NBUF:2 * NBUF]
        tvu, tvi = scratch[2 * NBUF], scratch[2 * NBUF + 1]
        lane_v = scratch[2 * NBUF + 2]
        sis = scratch[2 * NBUF + 3:2 * NBUF + 3 + NBUF]
        sos = scratch[2 * NBUF + 3 + NBUF:2 * NBUF + 3 + 2 * NBUF]
        _detr_body(ut, it, lane_hbm, scr, tvs, ovs, tvu, tvi, lane_v, sis, sos)

    detr = pl.kernel(
        detr_body,
        out_type=jax.ShapeDtypeStruct((M, 128), jnp.float32),
        mesh=mesh,
        compiler_params=params,
        scratch_types=(
            [pltpu.VMEM((D, 256), jnp.float32) for _ in range(NBUF)]
            + [pltpu.VMEM((64, 128), jnp.float32) for _ in range(NBUF)]
            + [pltpu.VMEM((D, WU_TAIL), jnp.float32),
               pltpu.VMEM((D, WI_TAIL), jnp.float32),
               pltpu.VMEM((L,), jnp.int32)]
            + [pltpu.SemaphoreType.DMA for _ in range(2 * NBUF)]
        ),
    )

    def gdot_body(scr, u, ii, ij, lane_hbm, oi, oj, *scratch):
        uidx_v, iidx_v, jidx_v, urow_v, irow_v, jrow_v = scratch[0:6]
        bufs = (scratch[6:9], scratch[9:12])
        oi_v, oj_v = scratch[12], scratch[13]
        lane_v = scratch[14]
        sgs = scratch[15:17]
        _gdot_body(scr, u, ii, ij, lane_hbm, oi, oj,
                   uidx_v, iidx_v, jidx_v, urow_v, irow_v, jrow_v,
                   bufs, oi_v, oj_v, lane_v, sgs)

    gdot = pl.kernel(
        gdot_body,
        out_type=(jax.ShapeDtypeStruct((B,), jnp.float32),
                  jax.ShapeDtypeStruct((B,), jnp.float32)),
        mesh=mesh,
        compiler_params=params,
        scratch_types=(
            [pltpu.VMEM((BPW,), jnp.int32) for _ in range(6)]
            + [pltpu.VMEM((CH, 128), jnp.float32) for _ in range(6)]
            + [pltpu.VMEM((BPW,), jnp.float32) for _ in range(2)]
            + [pltpu.VMEM((L,), jnp.int32)]
            + [pltpu.SemaphoreType.DMA for _ in range(2)]
        ),
    )
    lane_arr = jnp.arange(L, dtype=jnp.int32)
    scratch = detr(embed_user_weight.T, embed_item_weight.T, lane_arr)
    return gdot(scratch, user, item_i, item_j, lane_arr)


def kernel(user, item_i, item_j, embed_user_weight, embed_item_weight):
    return _bpr(user.astype(jnp.int32), item_i.astype(jnp.int32),
                item_j.astype(jnp.int32), embed_user_weight, embed_item_weight)
